# initial kernel scaffold (unmeasured)
import jax
import jax.numpy as jnp
from jax import lax
from jax.experimental import pallas as pl
from jax.experimental.pallas import tpu as pltpu

N_DEV = 8
M_PER = 1024
K = 8192
N_PER = 512
HALF = 256
N_STEPS = 2 * N_DEV
N_SLOTS = 4


def _gelu(y):
    c = 0.7978845608028654
    return 0.5 * y * (1.0 + jnp.tanh(c * (y + 0.044715 * y * y * y)))


def kernel(x, w_mat):
    assert x.shape == (M_PER, K), x.shape
    assert w_mat.shape == (K, N_DEV * N_PER), w_mat.shape

    def body(x_ref, w_ref, out_ref, w_buf, send_buf,
             w_sems, send_sems, recv_sems, local_sem):
        p = lax.axis_index("i")

        def w_dma(t):
            d, h = divmod(t, 2)
            j = lax.rem(p + d, N_DEV)
            off = j * N_PER + h * HALF
            return pltpu.make_async_copy(
                w_ref.at[:, pl.ds(off, HALF)],
                w_buf.at[t % 2],
                w_sems.at[t % 2],
            )

        def local_cp():
            return pltpu.make_async_copy(
                send_buf.at[0],
                out_ref.at[pl.ds(p * M_PER, M_PER), :],
                local_sem,
            )

        def send_rdma(d):
            j = lax.rem(p + d, N_DEV)
            return pltpu.make_async_remote_copy(
                src_ref=send_buf.at[d % N_SLOTS],
                dst_ref=out_ref.at[pl.ds(p * M_PER, M_PER), :],
                send_sem=send_sems.at[d % N_SLOTS],
                recv_sem=recv_sems.at[d],
                device_id=(j,),
                device_id_type=pl.DeviceIdType.MESH,
            )

        w_dma(0).start()
        w_dma(1).start()

        for t in range(N_STEPS):
            d, h = divmod(t, 2)
            slot = d % N_SLOTS
            if h == 0 and d >= N_SLOTS:
                prev = d - N_SLOTS
                if prev == 0:
                    local_cp().wait()
                else:
                    send_rdma(prev).wait_send()
            w_dma(t).wait()
            y = jnp.dot(x_ref[:, :], w_buf[t % 2],
                        preferred_element_type=jnp.float32)
            send_buf[slot, :, pl.ds(h * HALF, HALF)] = _gelu(y)
            if t + 2 < N_STEPS:
                w_dma(t + 2).start()
            if h == 1:
                if d == 0:
                    local_cp().start()
                else:
                    send_rdma(d).start()

        for d in range(N_DEV - N_SLOTS, N_DEV):
            send_rdma(d).wait_send()

        for dd in range(1, N_DEV):
            src_dev = lax.rem(p - dd + N_DEV, N_DEV)
            pltpu.make_async_remote_copy(
                src_ref=send_buf.at[0],
                dst_ref=out_ref.at[pl.ds(src_dev * M_PER, M_PER), :],
                send_sem=send_sems.at[0],
                recv_sem=recv_sems.at[dd],
                device_id=(p,),
                device_id_type=pl.DeviceIdType.MESH,
            ).wait_recv()

    return pl.pallas_call(
        body,
        out_shape=jax.ShapeDtypeStruct((N_DEV * M_PER, N_PER), jnp.float32),
        in_specs=[
            pl.BlockSpec(memory_space=pltpu.MemorySpace.VMEM),
            pl.BlockSpec(memory_space=pltpu.MemorySpace.HBM),
        ],
        out_specs=pl.BlockSpec(memory_space=pltpu.MemorySpace.HBM),
        scratch_shapes=[
            pltpu.VMEM((2, K, HALF), jnp.float32),
            pltpu.VMEM((N_SLOTS, M_PER, N_PER), jnp.float32),
            pltpu.SemaphoreType.DMA((2,)),
            pltpu.SemaphoreType.DMA((N_SLOTS,)),
            pltpu.SemaphoreType.DMA((N_DEV,)),
            pltpu.SemaphoreType.DMA,
        ],
        compiler_params=pltpu.CompilerParams(
            has_side_effects=True,
        ),
    )(x, w_mat)


# baseline (device time: 214635 ns/iter reference)
import jax
import jax.numpy as jnp
from jax import lax
from jax.experimental import pallas as pl
from jax.experimental.pallas import tpu as pltpu

N_DEV = 8
M_PER = 1024
K = 8192
N_PER = 512
HALF = 256
N_STEPS = 2 * N_DEV
N_SLOTS = 4


def _gelu(y):
    c = 0.7978845608028654
    return 0.5 * y * (1.0 + jnp.tanh(c * (y + 0.044715 * y * y * y)))


def kernel(x, w_mat):
    assert x.shape == (M_PER, K), x.shape
    assert w_mat.shape == (K, N_DEV * N_PER), w_mat.shape

    def body(x_ref, w_ref, out_ref, w_buf, send_buf,
             w_sems, send_sems, recv_sems, local_sem):
        p = lax.axis_index("i")

        def w_dma(t):
            d, h = divmod(t, 2)
            j = lax.rem(p + d, N_DEV)
            off = j * N_PER + h * HALF
            return pltpu.make_async_copy(
                w_ref.at[:, pl.ds(off, HALF)],
                w_buf.at[t % 2],
                w_sems.at[t % 2],
            )

        def local_cp():
            return pltpu.make_async_copy(
                send_buf.at[0],
                out_ref.at[pl.ds(p * M_PER, M_PER), :],
                local_sem,
            )

        def send_rdma(d):
            j = lax.rem(p + d, N_DEV)
            return pltpu.make_async_remote_copy(
                src_ref=send_buf.at[d % N_SLOTS],
                dst_ref=out_ref.at[pl.ds(p * M_PER, M_PER), :],
                send_sem=send_sems.at[d % N_SLOTS],
                recv_sem=recv_sems.at[d],
                device_id=(j,),
                device_id_type=pl.DeviceIdType.MESH,
            )

        w_dma(0).start()
        w_dma(1).start()

        for t in range(N_STEPS):
            d, h = divmod(t, 2)
            slot = d % N_SLOTS
            if h == 0 and d >= N_SLOTS:
                prev = d - N_SLOTS
                if prev == 0:
                    local_cp().wait()
                else:
                    send_rdma(prev).wait_send()
            w_dma(t).wait()
            y = jnp.dot(x_ref[:, :], w_buf[t % 2],
                        preferred_element_type=jnp.float32)
            send_buf[slot, :, pl.ds(h * HALF, HALF)] = _gelu(y)
            if t + 2 < N_STEPS:
                w_dma(t + 2).start()
            if h == 1:
                if d == 0:
                    local_cp().start()
                else:
                    send_rdma(d).start()

        for d in range(N_DEV - N_SLOTS, N_DEV):
            send_rdma(d).wait_send()

        for dd in range(1, N_DEV):
            src_dev = lax.rem(p - dd + N_DEV, N_DEV)
            pltpu.make_async_remote_copy(
                src_ref=send_buf.at[0],
                dst_ref=out_ref.at[pl.ds(src_dev * M_PER, M_PER), :],
                send_sem=send_sems.at[0],
                recv_sem=recv_sems.at[dd],
                device_id=(p,),
                device_id_type=pl.DeviceIdType.MESH,
            ).wait_recv()

    return pl.pallas_call(
        body,
        out_shape=jax.ShapeDtypeStruct((N_DEV * M_PER, N_PER), jnp.float32),
        in_specs=[
            pl.BlockSpec(memory_space=pltpu.MemorySpace.VMEM),
            pl.BlockSpec(memory_space=pltpu.MemorySpace.HBM),
        ],
        out_specs=pl.BlockSpec(memory_space=pltpu.MemorySpace.HBM),
        scratch_shapes=[
            pltpu.VMEM((2, K, HALF), jnp.float32),
            pltpu.VMEM((N_SLOTS, M_PER, N_PER), jnp.float32),
            pltpu.SemaphoreType.DMA((2,)),
            pltpu.SemaphoreType.DMA((N_SLOTS,)),
            pltpu.SemaphoreType.DMA((N_DEV,)),
            pltpu.SemaphoreType.DMA,
        ],
        compiler_params=pltpu.CompilerParams(
            has_side_effects=True,
            vmem_limit_bytes=62 * 1024 * 1024,
        ),
    )(x, w_mat)
